# Initial kernel scaffold; baseline (speedup 1.0000x reference)
import jax
import jax.numpy as jnp
from jax.experimental import pallas as pl


def kernel(x, table, bias, offsets):
    raise NotImplementedError("write your pallas kernel here")


# speedup vs baseline: 1.0100x; 1.0100x over previous
import jax
import jax.numpy as jnp
from jax import lax
from jax.experimental import pallas as pl
from jax.experimental.pallas import tpu as pltpu
from jax.experimental.pallas import tpu_sc as plsc

_BATCH = 4096
_FIELDS = 26
_LANES = 16
_NUM_CORES = 2
_NUM_SUBCORES = 16
_NUM_WORKERS = _NUM_CORES * _NUM_SUBCORES
_BPW = _BATCH // _NUM_WORKERS
_CHUNKS = _BPW // _LANES
_SLAB = _BPW * _FIELDS


def _body(x_hbm, off_hbm, bias_hbm, table_hbm, out_hbm,
          xv, off_v, bias_v, idx2d, val2d, acc_v, sem_g):
    wid = lax.axis_index("s") * _NUM_CORES + lax.axis_index("c")
    base = wid * _BPW

    pltpu.sync_copy(x_hbm.at[pl.ds(base * _FIELDS, _SLAB)], xv)
    pltpu.sync_copy(off_hbm, off_v)
    pltpu.sync_copy(bias_hbm, bias_v)

    iota26 = lax.iota(jnp.int32, _LANES) * _FIELDS

    for f in range(_FIELDS):
        off_b = off_v[f, :]
        for c in range(_CHUNKS):
            iv = iota26 + (c * _LANES * _FIELDS + f)
            xvals = plsc.load_gather(xv, [iv])
            idx2d[f, pl.ds(c * _LANES, _LANES)] = xvals + off_b

    copies = []
    for f in range(_FIELDS):
        cp = pltpu.make_async_copy(table_hbm.at[idx2d.at[f]], val2d.at[f],
                                   sem_g)
        cp.start()
        copies.append(cp)
    for cp in copies:
        cp.wait()

    bias_vec = bias_v[...]
    for c in range(_CHUNKS):
        acc = bias_vec
        for f in range(_FIELDS):
            acc = acc + val2d[f, pl.ds(c * _LANES, _LANES)]
        acc_v[pl.ds(c * _LANES, _LANES)] = acc

    pltpu.sync_copy(acc_v, out_hbm.at[pl.ds(base, _BPW)])


@jax.jit
def kernel(x, table, bias, offsets):
    x_flat = x.reshape(-1)
    table_flat = table.reshape(-1)
    bias_b = jnp.broadcast_to(bias.astype(jnp.float32), (_LANES,))
    off_b2d = jnp.broadcast_to(
        offsets.astype(jnp.int32)[:, None], (_FIELDS, _LANES))
    mesh = plsc.VectorSubcoreMesh(core_axis_name="c", subcore_axis_name="s",
                                  num_cores=_NUM_CORES,
                                  num_subcores=_NUM_SUBCORES)
    run = pl.kernel(
        _body,
        out_type=jax.ShapeDtypeStruct((_BATCH,), jnp.float32),
        mesh=mesh,
        compiler_params=pltpu.CompilerParams(needs_layout_passes=False),
        scratch_types=[
            pltpu.VMEM((_SLAB,), jnp.int32),
            pltpu.VMEM((_FIELDS, _LANES), jnp.int32),
            pltpu.VMEM((_LANES,), jnp.float32),
            pltpu.VMEM((_FIELDS, _BPW), jnp.int32),
            pltpu.VMEM((_FIELDS, _BPW), jnp.float32),
            pltpu.VMEM((_BPW,), jnp.float32),
            pltpu.SemaphoreType.DMA,
        ],
    )
    out = run(x_flat, off_b2d, bias_b, table_flat)
    return out.reshape(_BATCH, 1)
